# initial kernel scaffold (unmeasured)
import jax
import jax.numpy as jnp
from jax import lax
from jax.experimental import pallas as pl
from jax.experimental.pallas import tpu as pltpu

B, H, W, C = 2, 256, 256, 128
GH, GW = 512, 512
EPS = 1e-5

SBR = 32
SNB = H // SBR
BR = 16
NB = H // BR

_CP = getattr(pltpu, "CompilerParams", None) or pltpu.TPUCompilerParams
_ANY = getattr(pltpu, "ANY", None) or pltpu.MemorySpace.ANY
_MESH = pl.DeviceIdType.MESH


def _halo_exchange(x):

    def body(x_hbm, col_out, row_out, send_col, send_row, ext_row,
             local_sems, send_sems, recv_sems):
        mx = lax.axis_index("x")
        my = lax.axis_index("y")

        bar = pltpu.get_barrier_semaphore()
        pl.semaphore_signal(bar, inc=1, device_id=(1 - mx, my),
                            device_id_type=_MESH)
        pl.semaphore_signal(bar, inc=1, device_id=(mx, 1 - my),
                            device_id_type=_MESH)
        pl.semaphore_wait(bar, 2)

        cy = jnp.where(my == 0, W - 1, 0)
        rx = jnp.where(mx == 0, H - 1, 0)

        c_col = pltpu.make_async_copy(
            x_hbm.at[:, :, pl.ds(cy, 1), :], send_col, local_sems.at[0])
        c_col.start()
        c_row = pltpu.make_async_copy(
            x_hbm.at[:, pl.ds(rx, 1), :, :], send_row, local_sems.at[1])
        c_row.start()

        c_col.wait()
        colx = pltpu.make_async_remote_copy(
            src_ref=send_col, dst_ref=col_out,
            send_sem=send_sems.at[0], recv_sem=recv_sems.at[0],
            device_id=(mx, 1 - my), device_id_type=_MESH)
        colx.start()
        c_row.wait()
        colx.wait()

        rowv = send_row[...]
        colv = col_out[...]
        colr = lax.dynamic_slice(colv, (0, rx, 0, 0), (B, 1, 1, C))
        left = jnp.where(my == 0, rowv[:, :, 0:1, :], colr)
        right = jnp.where(my == 0, colr, rowv[:, :, W - 1:W, :])
        ext_row[...] = jnp.concatenate([left, rowv, right], axis=2)

        rowx = pltpu.make_async_remote_copy(
            src_ref=ext_row, dst_ref=row_out,
            send_sem=send_sems.at[1], recv_sem=recv_sems.at[1],
            device_id=(1 - mx, my), device_id_type=_MESH)
        rowx.start()
        rowx.wait()

    return pl.pallas_call(
        body,
        out_shape=(
            jax.ShapeDtypeStruct((B, H, 1, C), jnp.float32),
            jax.ShapeDtypeStruct((B, 1, W + 2, C), jnp.float32),
        ),
        in_specs=[pl.BlockSpec(memory_space=_ANY)],
        out_specs=(
            pl.BlockSpec(memory_space=pltpu.VMEM),
            pl.BlockSpec(memory_space=pltpu.VMEM),
        ),
        scratch_shapes=[
            pltpu.VMEM((B, H, 1, C), jnp.float32),
            pltpu.VMEM((B, 1, W, C), jnp.float32),
            pltpu.VMEM((B, 1, W + 2, C), jnp.float32),
            pltpu.SemaphoreType.DMA((2,)),
            pltpu.SemaphoreType.DMA((2,)),
            pltpu.SemaphoreType.DMA((2,)),
        ],
        compiler_params=_CP(collective_id=0),
    )(x)


def _stats_allreduce(x):

    def body(x_ref, out_ref, acc, sbuf, rxbuf, sbuf2, rybuf,
             send_sems, recv_sems):
        i = pl.program_id(0)
        mx = lax.axis_index("x")
        my = lax.axis_index("y")
        bar = pltpu.get_barrier_semaphore()

        @pl.when(i == 0)
        def _():
            pl.semaphore_signal(bar, inc=1, device_id=(1 - mx, my),
                                device_id_type=_MESH)
            pl.semaphore_signal(bar, inc=1, device_id=(mx, 1 - my),
                                device_id_type=_MESH)
            pl.semaphore_wait(bar, 2)
            acc[...] = jnp.zeros_like(acc)

        xb = x_ref[...]
        s = jnp.sum(xb, axis=(1, 2))
        s2 = jnp.sum(xb * xb, axis=(1, 2))
        acc[...] = acc[...] + jnp.stack([s, s2], axis=0)

        @pl.when(i == SNB - 1)
        def _():
            sbuf[...] = acc[...]
            rdx = pltpu.make_async_remote_copy(
                src_ref=sbuf, dst_ref=rxbuf,
                send_sem=send_sems.at[0], recv_sem=recv_sems.at[0],
                device_id=(1 - mx, my), device_id_type=_MESH)
            rdx.start()
            rdx.wait()
            sbuf2[...] = sbuf[...] + rxbuf[...]
            rdy = pltpu.make_async_remote_copy(
                src_ref=sbuf2, dst_ref=rybuf,
                send_sem=send_sems.at[1], recv_sem=recv_sems.at[1],
                device_id=(mx, 1 - my), device_id_type=_MESH)
            rdy.start()
            rdy.wait()
            tot = sbuf2[...] + rybuf[...]
            n = float(GH * GW)
            mean = tot[0] / n
            var = tot[1] / n - mean * mean
            rstd = lax.rsqrt(var + EPS)
            out_ref[...] = jnp.stack([mean, rstd], axis=0)

    return pl.pallas_call(
        body,
        grid=(SNB,),
        out_shape=jax.ShapeDtypeStruct((2, B, C), jnp.float32),
        in_specs=[pl.BlockSpec((B, SBR, W, C), lambda i: (0, i, 0, 0))],
        out_specs=pl.BlockSpec((2, B, C), lambda i: (0, 0, 0)),
        scratch_shapes=[
            pltpu.VMEM((2, B, C), jnp.float32),
            pltpu.VMEM((2, B, C), jnp.float32),
            pltpu.VMEM((2, B, C), jnp.float32),
            pltpu.VMEM((2, B, C), jnp.float32),
            pltpu.VMEM((2, B, C), jnp.float32),
            pltpu.SemaphoreType.DMA((2,)),
            pltpu.SemaphoreType.DMA((2,)),
        ],
        compiler_params=_CP(
            collective_id=1, dimension_semantics=("arbitrary",)),
    )(x)


def _main(x, k, Wp, stats, top_seam, bot_seam, left_cols, right_cols):

    def body(x_ref, k_ref, wp_ref, st_ref, ts_ref, bs_ref, lc_ref, rc_ref,
             o_ref):
        st = st_ref[...]
        mean = st[0]
        rstd = st[1]
        mb = mean[:, None, None, :]
        rb = rstd[:, None, None, :]

        xb = x_ref[...]
        h = (xb - mb) * rb
        lc = (lc_ref[...] - mean[:, None, :]) * rstd[:, None, :]
        rc = (rc_ref[...] - mean[:, None, :]) * rstd[:, None, :]
        ts = (ts_ref[0] - mean[:, None, :]) * rstd[:, None, :]
        bs = (bs_ref[0] - mean[:, None, :]) * rstd[:, None, :]

        center = jnp.concatenate(
            [lc[:, :, None, :], h, rc[:, :, None, :]], axis=2)
        padded = jnp.concatenate(
            [ts[:, None, :, :], center, bs[:, None, :, :]], axis=1)

        kv = k_ref[...]
        conv = jnp.zeros_like(xb)
        for di in range(3):
            for dj in range(3):
                conv = conv + (padded[:, di:di + BR, dj:dj + W, :]
                               * kv[di, dj][None, None, None, :])
        a = conv * jax.nn.sigmoid(conv)
        o = jnp.dot(a.reshape(B * BR * W, C), wp_ref[...],
                    preferred_element_type=jnp.float32)
        o_ref[...] = xb + o.reshape(B, BR, W, C)

    return pl.pallas_call(
        body,
        grid=(NB,),
        out_shape=jax.ShapeDtypeStruct((B, H, W, C), jnp.float32),
        in_specs=[
            pl.BlockSpec((B, BR, W, C), lambda i: (0, i, 0, 0)),
            pl.BlockSpec((3, 3, C), lambda i: (0, 0, 0)),
            pl.BlockSpec((C, C), lambda i: (0, 0)),
            pl.BlockSpec((2, B, C), lambda i: (0, 0, 0)),
            pl.BlockSpec((1, B, W + 2, C), lambda i: (i, 0, 0, 0)),
            pl.BlockSpec((1, B, W + 2, C), lambda i: (i, 0, 0, 0)),
            pl.BlockSpec((B, BR, C), lambda i: (0, i, 0)),
            pl.BlockSpec((B, BR, C), lambda i: (0, i, 0)),
        ],
        out_specs=pl.BlockSpec((B, BR, W, C), lambda i: (0, i, 0, 0)),
        compiler_params=_CP(dimension_semantics=("arbitrary",)),
    )(x, k, Wp, stats, top_seam, bot_seam, left_cols, right_cols)


def kernel(x, k, Wp):
    mx = lax.axis_index("x")
    my = lax.axis_index("y")

    col_halo, row_halo = _halo_exchange(x)
    stats = _stats_allreduce(x)

    ch = col_halo[:, :, 0, :]
    rh = row_halo[:, 0, :, :]

    top_idx = tuple(max(i * BR - 1, 0) for i in range(NB))
    bot_idx = tuple(min((i + 1) * BR, H - 1) for i in range(NB))
    tops = jnp.moveaxis(x[:, top_idx, :, :], 1, 0)
    bots = jnp.moveaxis(x[:, bot_idx, :, :], 1, 0)
    ch_t = jnp.moveaxis(ch[:, top_idx, :], 1, 0)[:, :, None, :]
    ch_b = jnp.moveaxis(ch[:, bot_idx, :], 1, 0)[:, :, None, :]

    is_y0 = my == 0
    lt = jnp.where(is_y0, tops[:, :, 0:1, :], ch_t)
    rt = jnp.where(is_y0, ch_t, tops[:, :, W - 1:W, :])
    top_seam = jnp.concatenate([lt, tops, rt], axis=2)
    lb = jnp.where(is_y0, bots[:, :, 0:1, :], ch_b)
    rb = jnp.where(is_y0, ch_b, bots[:, :, W - 1:W, :])
    bot_seam = jnp.concatenate([lb, bots, rb], axis=2)

    top_seam = top_seam.at[0].set(jnp.where(mx == 1, rh, top_seam[0]))
    bot_seam = bot_seam.at[NB - 1].set(
        jnp.where(mx == 0, rh, bot_seam[NB - 1]))

    left_cols = jnp.where(is_y0, x[:, :, 0, :], ch)
    right_cols = jnp.where(is_y0, ch, x[:, :, W - 1, :])

    return _main(x, k, Wp, stats, top_seam, bot_seam, left_cols, right_cols)


# baseline (device time: 156569 ns/iter reference)
import jax
import jax.numpy as jnp
from jax import lax
from jax.experimental import pallas as pl
from jax.experimental.pallas import tpu as pltpu

B, H, W, C = 2, 256, 256, 128
GH, GW = 512, 512
EPS = 1e-5

SBR = 32
SNB = H // SBR
BR = 16
NB = H // BR

_CP = getattr(pltpu, "CompilerParams", None) or pltpu.TPUCompilerParams
_ANY = pl.ANY
_MESH = pl.DeviceIdType.MESH


def _halo_exchange(x):

    def body(x_hbm, col_out, row_out, send_col, send_row, ext_row,
             local_sems, send_sems, recv_sems):
        mx = lax.axis_index("x")
        my = lax.axis_index("y")

        bar = pltpu.get_barrier_semaphore()
        pl.semaphore_signal(bar, inc=1, device_id=(1 - mx, my),
                            device_id_type=_MESH)
        pl.semaphore_signal(bar, inc=1, device_id=(mx, 1 - my),
                            device_id_type=_MESH)
        pl.semaphore_wait(bar, 2)

        cy = jnp.where(my == 0, W - 1, 0)
        rx = jnp.where(mx == 0, H - 1, 0)

        c_col = pltpu.make_async_copy(
            x_hbm.at[:, :, pl.ds(cy, 1), :], send_col, local_sems.at[0])
        c_col.start()
        c_row = pltpu.make_async_copy(
            x_hbm.at[:, pl.ds(rx, 1), :, :], send_row, local_sems.at[1])
        c_row.start()

        c_col.wait()
        colx = pltpu.make_async_remote_copy(
            src_ref=send_col, dst_ref=col_out,
            send_sem=send_sems.at[0], recv_sem=recv_sems.at[0],
            device_id=(mx, 1 - my), device_id_type=_MESH)
        colx.start()
        c_row.wait()
        colx.wait()

        rowv = send_row[...]
        colr = col_out[:, pl.ds(rx, 1), :, :]
        left = jnp.where(my == 0, rowv[:, :, 0:1, :], colr)
        right = jnp.where(my == 0, colr, rowv[:, :, W - 1:W, :])
        ext_row[...] = jnp.concatenate([left, rowv, right], axis=2)

        rowx = pltpu.make_async_remote_copy(
            src_ref=ext_row, dst_ref=row_out,
            send_sem=send_sems.at[1], recv_sem=recv_sems.at[1],
            device_id=(1 - mx, my), device_id_type=_MESH)
        rowx.start()
        rowx.wait()

    return pl.pallas_call(
        body,
        out_shape=(
            jax.ShapeDtypeStruct((B, H, 1, C), jnp.float32),
            jax.ShapeDtypeStruct((B, 1, W + 2, C), jnp.float32),
        ),
        in_specs=[pl.BlockSpec(memory_space=_ANY)],
        out_specs=(
            pl.BlockSpec(memory_space=pltpu.VMEM),
            pl.BlockSpec(memory_space=pltpu.VMEM),
        ),
        scratch_shapes=[
            pltpu.VMEM((B, H, 1, C), jnp.float32),
            pltpu.VMEM((B, 1, W, C), jnp.float32),
            pltpu.VMEM((B, 1, W + 2, C), jnp.float32),
            pltpu.SemaphoreType.DMA((2,)),
            pltpu.SemaphoreType.DMA((2,)),
            pltpu.SemaphoreType.DMA((2,)),
        ],
        compiler_params=_CP(collective_id=0),
    )(x)


def _stats_allreduce(x):

    def body(x_ref, out_ref, acc, sbuf, rxbuf, sbuf2, rybuf,
             send_sems, recv_sems):
        i = pl.program_id(0)
        mx = lax.axis_index("x")
        my = lax.axis_index("y")
        bar = pltpu.get_barrier_semaphore()

        @pl.when(i == 0)
        def _():
            pl.semaphore_signal(bar, inc=1, device_id=(1 - mx, my),
                                device_id_type=_MESH)
            pl.semaphore_signal(bar, inc=1, device_id=(mx, 1 - my),
                                device_id_type=_MESH)
            pl.semaphore_wait(bar, 2)
            acc[...] = jnp.zeros_like(acc)

        xb = x_ref[...]
        s = jnp.sum(xb, axis=(1, 2))
        s2 = jnp.sum(xb * xb, axis=(1, 2))
        acc[...] = acc[...] + jnp.stack([s, s2], axis=0)

        @pl.when(i == SNB - 1)
        def _():
            sbuf[...] = acc[...]
            rdx = pltpu.make_async_remote_copy(
                src_ref=sbuf, dst_ref=rxbuf,
                send_sem=send_sems.at[0], recv_sem=recv_sems.at[0],
                device_id=(1 - mx, my), device_id_type=_MESH)
            rdx.start()
            rdx.wait()
            sbuf2[...] = sbuf[...] + rxbuf[...]
            rdy = pltpu.make_async_remote_copy(
                src_ref=sbuf2, dst_ref=rybuf,
                send_sem=send_sems.at[1], recv_sem=recv_sems.at[1],
                device_id=(mx, 1 - my), device_id_type=_MESH)
            rdy.start()
            rdy.wait()
            tot = sbuf2[...] + rybuf[...]
            n = float(GH * GW)
            mean = tot[0] / n
            var = tot[1] / n - mean * mean
            rstd = lax.rsqrt(var + EPS)
            out_ref[...] = jnp.stack([mean, rstd], axis=0)

    return pl.pallas_call(
        body,
        grid=(SNB,),
        out_shape=jax.ShapeDtypeStruct((2, B, C), jnp.float32),
        in_specs=[pl.BlockSpec((B, SBR, W, C), lambda i: (0, i, 0, 0))],
        out_specs=pl.BlockSpec((2, B, C), lambda i: (0, 0, 0)),
        scratch_shapes=[
            pltpu.VMEM((2, B, C), jnp.float32),
            pltpu.VMEM((2, B, C), jnp.float32),
            pltpu.VMEM((2, B, C), jnp.float32),
            pltpu.VMEM((2, B, C), jnp.float32),
            pltpu.VMEM((2, B, C), jnp.float32),
            pltpu.SemaphoreType.DMA((2,)),
            pltpu.SemaphoreType.DMA((2,)),
        ],
        compiler_params=_CP(
            collective_id=1, dimension_semantics=("arbitrary",)),
    )(x)


def _main(x, k, Wp, stats, top_seam, bot_seam, left_cols, right_cols):

    def body(x_ref, k_ref, wp_ref, st_ref, ts_ref, bs_ref, lc_ref, rc_ref,
             o_ref):
        st = st_ref[...]
        mean = st[0]
        rstd = st[1]
        mb = mean[:, None, None, :]
        rb = rstd[:, None, None, :]

        xb = x_ref[...]
        h = (xb - mb) * rb
        lc = (lc_ref[...] - mean[:, None, :]) * rstd[:, None, :]
        rc = (rc_ref[...] - mean[:, None, :]) * rstd[:, None, :]
        ts = (ts_ref[0] - mean[:, None, :]) * rstd[:, None, :]
        bs = (bs_ref[0] - mean[:, None, :]) * rstd[:, None, :]

        center = jnp.concatenate(
            [lc[:, :, None, :], h, rc[:, :, None, :]], axis=2)
        padded = jnp.concatenate(
            [ts[:, None, :, :], center, bs[:, None, :, :]], axis=1)

        kv = k_ref[...]
        conv = jnp.zeros_like(xb)
        for di in range(3):
            for dj in range(3):
                conv = conv + (padded[:, di:di + BR, dj:dj + W, :]
                               * kv[di, dj][None, None, None, :])
        a = conv * jax.nn.sigmoid(conv)
        o = jnp.dot(a.reshape(B * BR * W, C), wp_ref[...],
                    preferred_element_type=jnp.float32)
        o_ref[...] = xb + o.reshape(B, BR, W, C)

    return pl.pallas_call(
        body,
        grid=(NB,),
        out_shape=jax.ShapeDtypeStruct((B, H, W, C), jnp.float32),
        in_specs=[
            pl.BlockSpec((B, BR, W, C), lambda i: (0, i, 0, 0)),
            pl.BlockSpec((3, 3, C), lambda i: (0, 0, 0)),
            pl.BlockSpec((C, C), lambda i: (0, 0)),
            pl.BlockSpec((2, B, C), lambda i: (0, 0, 0)),
            pl.BlockSpec((1, B, W + 2, C), lambda i: (i, 0, 0, 0)),
            pl.BlockSpec((1, B, W + 2, C), lambda i: (i, 0, 0, 0)),
            pl.BlockSpec((B, BR, C), lambda i: (0, i, 0)),
            pl.BlockSpec((B, BR, C), lambda i: (0, i, 0)),
        ],
        out_specs=pl.BlockSpec((B, BR, W, C), lambda i: (0, i, 0, 0)),
        compiler_params=_CP(dimension_semantics=("arbitrary",)),
    )(x, k, Wp, stats, top_seam, bot_seam, left_cols, right_cols)


def kernel(x, k, Wp):
    mx = lax.axis_index("x")
    my = lax.axis_index("y")

    col_halo, row_halo = _halo_exchange(x)
    stats = _stats_allreduce(x)

    ch = col_halo[:, :, 0, :]
    rh = row_halo[:, 0, :, :]

    top_idx = tuple(max(i * BR - 1, 0) for i in range(NB))
    bot_idx = tuple(min((i + 1) * BR, H - 1) for i in range(NB))
    tops = jnp.moveaxis(x[:, top_idx, :, :], 1, 0)
    bots = jnp.moveaxis(x[:, bot_idx, :, :], 1, 0)
    ch_t = jnp.moveaxis(ch[:, top_idx, :], 1, 0)[:, :, None, :]
    ch_b = jnp.moveaxis(ch[:, bot_idx, :], 1, 0)[:, :, None, :]

    is_y0 = my == 0
    lt = jnp.where(is_y0, tops[:, :, 0:1, :], ch_t)
    rt = jnp.where(is_y0, ch_t, tops[:, :, W - 1:W, :])
    top_seam = jnp.concatenate([lt, tops, rt], axis=2)
    lb = jnp.where(is_y0, bots[:, :, 0:1, :], ch_b)
    rb = jnp.where(is_y0, ch_b, bots[:, :, W - 1:W, :])
    bot_seam = jnp.concatenate([lb, bots, rb], axis=2)

    top_seam = top_seam.at[0].set(jnp.where(mx == 1, rh, top_seam[0]))
    bot_seam = bot_seam.at[NB - 1].set(
        jnp.where(mx == 0, rh, bot_seam[NB - 1]))

    left_cols = jnp.where(is_y0, x[:, :, 0, :], ch)
    right_cols = jnp.where(is_y0, ch, x[:, :, W - 1, :])

    return _main(x, k, Wp, stats, top_seam, bot_seam, left_cols, right_cols)
